# Initial kernel scaffold; baseline (speedup 1.0000x reference)
import functools
import numpy as np
import jax
import jax.numpy as jnp
from jax.experimental import pallas as pl
from jax.experimental.pallas import tpu as pltpu

D = 24
M = 16384
HOLO = 512
K = 32
SCALES = 4
IN = 512

_HIGH = jax.lax.Precision.HIGHEST


def _erf(x):
    a1, a2, a3, a4, a5 = (0.254829592, -0.284496736, 1.421413741,
                          -1.453152027, 1.061405429)
    p = 0.3275911
    s = jnp.sign(x)
    z = jnp.abs(x)
    t = 1.0 / (1.0 + p * z)
    poly = t * (a1 + t * (a2 + t * (a3 + t * (a4 + t * a5))))
    return s * (1.0 - poly * jnp.exp(-z * z))


def _gelu(x):
    return x * 0.5 * (1.0 + _erf(x * np.float32(1.0 / np.sqrt(2.0))))


def _ln(h, g, b):
    mu = jnp.mean(h, axis=-1, keepdims=True)
    v = jnp.mean((h - mu) ** 2, axis=-1, keepdims=True)
    return (h - mu) / jnp.sqrt(v + 1e-5) * g + b


def _cos_sin_2pi(u):
    t = (2.0 * np.pi) * u
    t2 = t * t
    ccoef = [1.0, -0.5, 1.0 / 24, -1.0 / 720, 1.0 / 40320,
             -1.0 / 3628800, 1.0 / 479001600, -1.0 / 87178291200]
    scoef = [1.0, -1.0 / 6, 1.0 / 120, -1.0 / 5040, 1.0 / 362880,
             -1.0 / 39916800, 1.0 / 6227020800]
    c = jnp.full_like(t, np.float32(ccoef[-1]))
    for a in ccoef[-2::-1]:
        c = c * t2 + np.float32(a)
    s = jnp.full_like(t, np.float32(scoef[-1]))
    for a in scoef[-2::-1]:
        s = s * t2 + np.float32(a)
    return c, s * t


def _s1_body(x_ref, Wp_ref, bp_ref, g1_ref, b1_ref, ricci_ref,
             Wkp_ref, bkp_ref, Wc_ref, Ws_ref,
             q_ref, kfre_ref, kfim_ref):
    x = x_ref[...]
    t = jnp.dot(x, Wp_ref[...], precision=_HIGH,
                preferred_element_type=jnp.float32) + bp_ref[...]
    h = _gelu(_ln(t, g1_ref[...], b1_ref[...]))
    hm = jnp.mean(h.reshape(h.shape[0], D, 3), axis=2)
    q_ref[...] = jnp.dot(hm, ricci_ref[...], precision=_HIGH,
                         preferred_element_type=jnp.float32)
    ph = jnp.dot(x, Wkp_ref[...], precision=_HIGH,
                 preferred_element_type=jnp.float32) + bkp_ref[...]
    sg = 1.0 / (1.0 + jnp.exp(-ph))
    u = sg - jnp.floor(sg + 0.5)
    c, s = _cos_sin_2pi(u)
    kfre_ref[...] = (jnp.dot(c, Wc_ref[...], precision=_HIGH,
                             preferred_element_type=jnp.float32)
                     - jnp.dot(s, Ws_ref[...], precision=_HIGH,
                               preferred_element_type=jnp.float32))
    kfim_ref[...] = (jnp.dot(c, Ws_ref[...], precision=_HIGH,
                             preferred_element_type=jnp.float32)
                     + jnp.dot(s, Wc_ref[...], precision=_HIGH,
                               preferred_element_type=jnp.float32))


def _s4_body(kfre_ref, kfim_ref, hre_ref, him_ref, A_ref, B_ref, b2_ref,
             g2_ref, be2_ref, out_ref):
    kr = kfre_ref[...]
    ki = kfim_ref[...]
    hr = hre_ref[...]
    hi = him_ref[...]
    rev = kr * hr + ki * hi
    imv = kr * hi - ki * hr
    r2 = (jnp.dot(rev, A_ref[...], precision=_HIGH,
                  preferred_element_type=jnp.float32)
          + jnp.dot(imv, B_ref[...], precision=_HIGH,
                    preferred_element_type=jnp.float32) + b2_ref[...])
    out_ref[...] = _gelu(_ln(r2, g2_ref[...], be2_ref[...]))


def kernel(x, keys, ricci, Wp, bp, ln1_g, ln1_b, fractal_w, Wkp, bkp,
           ent_key, holo_re, holo_im, Wro, bro, Wo, bo, ln2_g, ln2_b):
    B, S, _ = x.shape
    BS = B * S
    xf = x.reshape(BS, IN)

    fw = jax.nn.softmax(fractal_w)
    c_scale = jnp.sum(fw * (0.25 ** jnp.arange(SCALES, dtype=jnp.float32)))
    mj = jnp.arange(HOLO, dtype=jnp.float32)
    ang = (2.0 * np.pi / HOLO) * jnp.outer(mj, mj)
    Wc = jnp.cos(ent_key[None, :] - ang)
    Ws = jnp.sin(ent_key[None, :] - ang)
    Er = jnp.cos(ang) * (1.0 / HOLO)
    Ei = jnp.sin(ang) * (1.0 / HOLO)
    Wf = jnp.dot(Wro, Wo, precision=_HIGH)
    Wf_t, Wf_b = Wf[:HOLO], Wf[HOLO:]
    A2 = (jnp.dot(Er, Wf_t, precision=_HIGH)
          + jnp.dot(Ei, Wf_b, precision=_HIGH))
    B2 = (jnp.dot(Er, Wf_b, precision=_HIGH)
          - jnp.dot(Ei, Wf_t, precision=_HIGH))
    b2 = jnp.dot(bro, Wo, precision=_HIGH) + bo

    nblk = BS // 128
    s1 = pl.pallas_call(
        _s1_body,
        grid=(nblk,),
        in_specs=[
            pl.BlockSpec((128, IN), lambda i: (i, 0)),
            pl.BlockSpec((IN, 3 * D), lambda i: (0, 0)),
            pl.BlockSpec((3 * D,), lambda i: (0,)),
            pl.BlockSpec((3 * D,), lambda i: (0,)),
            pl.BlockSpec((3 * D,), lambda i: (0,)),
            pl.BlockSpec((D, D), lambda i: (0, 0)),
            pl.BlockSpec((IN, HOLO), lambda i: (0, 0)),
            pl.BlockSpec((HOLO,), lambda i: (0,)),
            pl.BlockSpec((HOLO, HOLO), lambda i: (0, 0)),
            pl.BlockSpec((HOLO, HOLO), lambda i: (0, 0)),
        ],
        out_specs=[
            pl.BlockSpec((128, D), lambda i: (i, 0)),
            pl.BlockSpec((128, HOLO), lambda i: (i, 0)),
            pl.BlockSpec((128, HOLO), lambda i: (i, 0)),
        ],
        out_shape=[
            jax.ShapeDtypeStruct((BS, D), jnp.float32),
            jax.ShapeDtypeStruct((BS, HOLO), jnp.float32),
            jax.ShapeDtypeStruct((BS, HOLO), jnp.float32),
        ],
    )(xf, Wp, bp, ln1_g, ln1_b, ricci, Wkp, bkp, Wc, Ws)
    q, kfre, kfim = s1

    q2 = jnp.sum(q * q, axis=-1)[:, None]
    k2 = jnp.sum(keys * keys, axis=-1)[None, :]
    qk = jnp.dot(q, keys.T, precision=_HIGH)
    scores = -c_scale * jnp.maximum(q2 + k2 - 2.0 * qk, 0.0)
    vals, idx = jax.lax.top_k(scores, K)
    wts = jax.nn.softmax(vals, axis=-1)

    hbar_re = jnp.einsum('nk,nkh->nh', wts, holo_re[idx])
    hbar_im = jnp.einsum('nk,nkh->nh', wts, holo_im[idx])

    out = pl.pallas_call(
        _s4_body,
        grid=(nblk,),
        in_specs=[
            pl.BlockSpec((128, HOLO), lambda i: (i, 0)),
            pl.BlockSpec((128, HOLO), lambda i: (i, 0)),
            pl.BlockSpec((128, HOLO), lambda i: (i, 0)),
            pl.BlockSpec((128, HOLO), lambda i: (i, 0)),
            pl.BlockSpec((HOLO, IN), lambda i: (0, 0)),
            pl.BlockSpec((HOLO, IN), lambda i: (0, 0)),
            pl.BlockSpec((IN,), lambda i: (0,)),
            pl.BlockSpec((IN,), lambda i: (0,)),
            pl.BlockSpec((IN,), lambda i: (0,)),
        ],
        out_specs=pl.BlockSpec((128, IN), lambda i: (i, 0)),
        out_shape=jax.ShapeDtypeStruct((BS, IN), jnp.float32),
    )(kfre, kfim, hbar_re, hbar_im, A2, B2, b2, ln2_g, ln2_b)
    return out.reshape(B, S, IN)


# speedup vs baseline: 2.2520x; 2.2520x over previous
import functools
import numpy as np
import jax
import jax.numpy as jnp
from jax.experimental import pallas as pl
from jax.experimental.pallas import tpu as pltpu

D = 24
M = 16384
HOLO = 512
K = 32
SCALES = 4
IN = 512

_HIGH = jax.lax.Precision.HIGHEST


def _erf(x):
    a1, a2, a3, a4, a5 = (0.254829592, -0.284496736, 1.421413741,
                          -1.453152027, 1.061405429)
    p = 0.3275911
    s = jnp.sign(x)
    z = jnp.abs(x)
    t = 1.0 / (1.0 + p * z)
    poly = t * (a1 + t * (a2 + t * (a3 + t * (a4 + t * a5))))
    return s * (1.0 - poly * jnp.exp(-z * z))


def _gelu(x):
    return x * 0.5 * (1.0 + _erf(x * np.float32(1.0 / np.sqrt(2.0))))


def _ln(h, g, b):
    mu = jnp.mean(h, axis=-1, keepdims=True)
    v = jnp.mean((h - mu) ** 2, axis=-1, keepdims=True)
    return (h - mu) / jnp.sqrt(v + 1e-5) * g + b


def _cos_sin_2pi(u):
    t = (2.0 * np.pi) * u
    t2 = t * t
    ccoef = [1.0, -0.5, 1.0 / 24, -1.0 / 720, 1.0 / 40320,
             -1.0 / 3628800, 1.0 / 479001600, -1.0 / 87178291200]
    scoef = [1.0, -1.0 / 6, 1.0 / 120, -1.0 / 5040, 1.0 / 362880,
             -1.0 / 39916800, 1.0 / 6227020800]
    c = jnp.full_like(t, np.float32(ccoef[-1]))
    for a in ccoef[-2::-1]:
        c = c * t2 + np.float32(a)
    s = jnp.full_like(t, np.float32(scoef[-1]))
    for a in scoef[-2::-1]:
        s = s * t2 + np.float32(a)
    return c, s * t


def _bdot(a, b):
    return jnp.dot(a.astype(jnp.bfloat16), b.astype(jnp.bfloat16),
                   preferred_element_type=jnp.float32)


def _s1_body(x_ref, Wp_ref, bp_ref, g1_ref, b1_ref, ricci_ref,
             Wkp_ref, bkp_ref, Wc_ref, Ws_ref, sel_ref,
             q_ref, kfre_ref, kfim_ref):
    x = x_ref[...]
    t = _bdot(x, Wp_ref[...]) + bp_ref[...]
    h = _gelu(_ln(t, g1_ref[...], b1_ref[...]))
    hb = h.astype(jnp.bfloat16)
    zsel = jnp.dot(hb, sel_ref[...].astype(jnp.bfloat16),
                   preferred_element_type=jnp.float32)
    rb = ricci_ref[...]
    y = (_bdot(zsel[:, :D], rb) + _bdot(zsel[:, D:2 * D], rb)
         + _bdot(zsel[:, 2 * D:], rb))
    q_ref[...] = y * np.float32(1.0 / 3.0)
    ph = _bdot(x, Wkp_ref[...]) + bkp_ref[...]
    sg = 1.0 / (1.0 + jnp.exp(-ph))
    u = sg - jnp.floor(sg + 0.5)
    c, s = _cos_sin_2pi(u)
    kfre_ref[...] = (jnp.dot(c, Wc_ref[...], precision=_HIGH,
                             preferred_element_type=jnp.float32)
                     - jnp.dot(s, Ws_ref[...], precision=_HIGH,
                               preferred_element_type=jnp.float32))
    kfim_ref[...] = (jnp.dot(c, Ws_ref[...], precision=_HIGH,
                             preferred_element_type=jnp.float32)
                     + jnp.dot(s, Wc_ref[...], precision=_HIGH,
                               preferred_element_type=jnp.float32))


def _s4_body(kfre_ref, kfim_ref, hre_ref, him_ref, A_ref, B_ref, b2_ref,
             g2_ref, be2_ref, out_ref):
    kr = kfre_ref[...]
    ki = kfim_ref[...]
    hr = hre_ref[...]
    hi = him_ref[...]
    rev = kr * hr + ki * hi
    imv = kr * hi - ki * hr
    r2 = (jnp.dot(rev, A_ref[...], precision=_HIGH,
                  preferred_element_type=jnp.float32)
          + jnp.dot(imv, B_ref[...], precision=_HIGH,
                    preferred_element_type=jnp.float32) + b2_ref[...])
    out_ref[...] = _gelu(_ln(r2, g2_ref[...], be2_ref[...]))


def kernel(x, keys, ricci, Wp, bp, ln1_g, ln1_b, fractal_w, Wkp, bkp,
           ent_key, holo_re, holo_im, Wro, bro, Wo, bo, ln2_g, ln2_b):
    B, S, _ = x.shape
    BS = B * S
    xf = x.reshape(BS, IN)

    fw = jax.nn.softmax(fractal_w)
    c_scale = jnp.sum(fw * (0.25 ** jnp.arange(SCALES, dtype=jnp.float32)))
    mj = jnp.arange(HOLO, dtype=jnp.float32)
    ang = (2.0 * np.pi / HOLO) * jnp.outer(mj, mj)
    Wc = jnp.cos(ent_key[None, :] - ang)
    Ws = jnp.sin(ent_key[None, :] - ang)
    Er = jnp.cos(ang) * (1.0 / HOLO)
    Ei = jnp.sin(ang) * (1.0 / HOLO)
    Wf = jnp.dot(Wro, Wo, precision=_HIGH)
    Wf_t, Wf_b = Wf[:HOLO], Wf[HOLO:]
    A2 = (jnp.dot(Er, Wf_t, precision=_HIGH)
          + jnp.dot(Ei, Wf_b, precision=_HIGH))
    B2 = (jnp.dot(Er, Wf_b, precision=_HIGH)
          - jnp.dot(Ei, Wf_t, precision=_HIGH))
    b2 = jnp.dot(bro, Wo, precision=_HIGH) + bo
    ej = np.arange(3 * D)
    sel_np = np.zeros((3 * D, 3 * D), np.float32)
    sel_np[ej, (ej % 3) * D + ej // 3] = 1.0
    sel = jnp.asarray(sel_np)

    nblk = BS // 128
    s1 = pl.pallas_call(
        _s1_body,
        grid=(nblk,),
        in_specs=[
            pl.BlockSpec((128, IN), lambda i: (i, 0)),
            pl.BlockSpec((IN, 3 * D), lambda i: (0, 0)),
            pl.BlockSpec((3 * D,), lambda i: (0,)),
            pl.BlockSpec((3 * D,), lambda i: (0,)),
            pl.BlockSpec((3 * D,), lambda i: (0,)),
            pl.BlockSpec((D, D), lambda i: (0, 0)),
            pl.BlockSpec((IN, HOLO), lambda i: (0, 0)),
            pl.BlockSpec((HOLO,), lambda i: (0,)),
            pl.BlockSpec((HOLO, HOLO), lambda i: (0, 0)),
            pl.BlockSpec((HOLO, HOLO), lambda i: (0, 0)),
            pl.BlockSpec((3 * D, 3 * D), lambda i: (0, 0)),
        ],
        out_specs=[
            pl.BlockSpec((128, D), lambda i: (i, 0)),
            pl.BlockSpec((128, HOLO), lambda i: (i, 0)),
            pl.BlockSpec((128, HOLO), lambda i: (i, 0)),
        ],
        out_shape=[
            jax.ShapeDtypeStruct((BS, D), jnp.float32),
            jax.ShapeDtypeStruct((BS, HOLO), jnp.float32),
            jax.ShapeDtypeStruct((BS, HOLO), jnp.float32),
        ],
    )(xf, Wp, bp, ln1_g, ln1_b, ricci, Wkp, bkp, Wc, Ws, sel)
    q, kfre, kfim = s1

    q2 = jnp.sum(q * q, axis=-1)[:, None]
    k2 = jnp.sum(keys * keys, axis=-1)[None, :]
    qk = jnp.dot(q.astype(jnp.bfloat16), keys.T.astype(jnp.bfloat16),
                 preferred_element_type=jnp.float32)
    scores = -c_scale * jnp.maximum(q2 + k2 - 2.0 * qk, 0.0)
    vals, idx = jax.lax.top_k(scores, K)
    wts = jax.nn.softmax(vals, axis=-1)

    hbar_re = jnp.einsum('nk,nkh->nh', wts, holo_re[idx])
    hbar_im = jnp.einsum('nk,nkh->nh', wts, holo_im[idx])

    out = pl.pallas_call(
        _s4_body,
        grid=(nblk,),
        in_specs=[
            pl.BlockSpec((128, HOLO), lambda i: (i, 0)),
            pl.BlockSpec((128, HOLO), lambda i: (i, 0)),
            pl.BlockSpec((128, HOLO), lambda i: (i, 0)),
            pl.BlockSpec((128, HOLO), lambda i: (i, 0)),
            pl.BlockSpec((HOLO, IN), lambda i: (0, 0)),
            pl.BlockSpec((HOLO, IN), lambda i: (0, 0)),
            pl.BlockSpec((IN,), lambda i: (0,)),
            pl.BlockSpec((IN,), lambda i: (0,)),
            pl.BlockSpec((IN,), lambda i: (0,)),
        ],
        out_specs=pl.BlockSpec((128, IN), lambda i: (i, 0)),
        out_shape=jax.ShapeDtypeStruct((BS, IN), jnp.float32),
    )(kfre, kfim, hbar_re, hbar_im, A2, B2, b2, ln2_g, ln2_b)
    return out.reshape(B, S, IN)
